# Initial kernel scaffold; baseline (speedup 1.0000x reference)
import jax
import jax.numpy as jnp
from jax.experimental import pallas as pl


def kernel(hidden_states, gate_w, w1, w3, w2):
    raise NotImplementedError("write your pallas kernel here")


# speedup vs baseline: 1.5470x; 1.5470x over previous
import functools

import jax
import jax.numpy as jnp
from jax.experimental import pallas as pl
from jax.experimental.pallas import tpu as pltpu

E = 64
H = 768
F = 2048
TOP_K = 2

T = 4096
FB = 512
TCHUNK = 1024


def _router_kernel(x_ref, gw_ref, logits_ref, wfull_ref):
    x = x_ref[...]
    gw = gw_ref[...]
    logits = jax.lax.dot_general(
        x, gw, (((1,), (1,)), ((), ())), preferred_element_type=jnp.float32)
    logits_ref[...] = logits
    m = jnp.max(logits, axis=1, keepdims=True)
    ex = jnp.exp(logits - m)
    rw = ex / jnp.sum(ex, axis=1, keepdims=True)
    lane = jax.lax.broadcasted_iota(jnp.int32, rw.shape, 1)
    v1 = jnp.max(rw, axis=1, keepdims=True)
    i1 = jnp.min(jnp.where(rw == v1, lane, E), axis=1, keepdims=True)
    rw2 = jnp.where(lane == i1, -jnp.inf, rw)
    v2 = jnp.max(rw2, axis=1, keepdims=True)
    i2 = jnp.min(jnp.where(rw2 == v2, lane, E), axis=1, keepdims=True)
    s = v1 + v2
    wfull_ref[...] = (jnp.where(lane == i1, v1 / s, 0.0)
                      + jnp.where(lane == i2, v2 / s, 0.0))


def _ffn_kernel(x_ref, w1_ref, w3_ref, w2_ref, wcol_ref, out_ref):
    e = pl.program_id(0)
    fb = pl.program_id(1)

    @pl.when(jnp.logical_and(e == 0, fb == 0))
    def _init():
        out_ref[...] = jnp.zeros_like(out_ref)

    w1 = w1_ref[0].astype(jnp.bfloat16)
    w3 = w3_ref[0].astype(jnp.bfloat16)
    w2 = w2_ref[0].astype(jnp.bfloat16)
    for c in range(T // TCHUNK):
        sl = pl.ds(c * TCHUNK, TCHUNK)
        xc = x_ref[sl, :].astype(jnp.bfloat16)
        h1 = jax.lax.dot_general(
            xc, w1, (((1,), (1,)), ((), ())), preferred_element_type=jnp.float32)
        h3 = jax.lax.dot_general(
            xc, w3, (((1,), (1,)), ((), ())), preferred_element_type=jnp.float32)
        hmid = (h1 * (1.0 / (1.0 + jnp.exp(-h1))) * h3).astype(jnp.bfloat16)
        y = jax.lax.dot_general(
            hmid, w2, (((1,), (1,)), ((), ())), preferred_element_type=jnp.float32)
        out_ref[sl, :] += y * wcol_ref[0, sl, :]


@jax.jit
def kernel(hidden_states, gate_w, w1, w3, w2):
    b, s, h = hidden_states.shape
    x = hidden_states.reshape(-1, h)

    logits, wfull = pl.pallas_call(
        _router_kernel,
        out_shape=(
            jax.ShapeDtypeStruct((T, E), jnp.float32),
            jax.ShapeDtypeStruct((T, E), jnp.float32),
        ),
    )(x, gate_w)

    final = pl.pallas_call(
        _ffn_kernel,
        grid=(E, F // FB),
        in_specs=[
            pl.BlockSpec((T, H), lambda e, fb: (0, 0)),
            pl.BlockSpec((1, FB, H), lambda e, fb: (e, fb, 0)),
            pl.BlockSpec((1, FB, H), lambda e, fb: (e, fb, 0)),
            pl.BlockSpec((1, H, FB), lambda e, fb: (e, 0, fb)),
            pl.BlockSpec((1, T, 1), lambda e, fb: (e, 0, 0)),
        ],
        out_specs=pl.BlockSpec((T, H), lambda e, fb: (0, 0)),
        out_shape=jax.ShapeDtypeStruct((T, H), jnp.float32),
    )(x, w1, w3, w2, wfull.T.reshape(E, T, 1))

    return (final, logits)


# speedup vs baseline: 5.4494x; 3.5226x over previous
import functools

import jax
import jax.numpy as jnp
from jax import lax
from jax.experimental import pallas as pl
from jax.experimental.pallas import tpu as pltpu
from jax.experimental.pallas import tpu_sc as plsc

E = 64
H = 768
F = 2048
TOP_K = 2

T = 4096
A = T * TOP_K
TM = 128
NUM_TILES = A // TM
G = NUM_TILES + E - 1
FB = 512

NC, NS = 2, 16
NW = NC * NS


def _router_kernel(x_ref, gw_ref, logits_ref, i1_ref, i2_ref, w1n_ref, w2n_ref):
    x = x_ref[...]
    gw = gw_ref[...]
    logits = lax.dot_general(
        x, gw, (((1,), (1,)), ((), ())), preferred_element_type=jnp.float32)
    logits_ref[...] = logits
    m = jnp.max(logits, axis=1, keepdims=True)
    ex = jnp.exp(logits - m)
    rw = ex / jnp.sum(ex, axis=1, keepdims=True)
    lane = lax.broadcasted_iota(jnp.int32, rw.shape, 1)
    v1 = jnp.max(rw, axis=1, keepdims=True)
    i1 = jnp.min(jnp.where(rw == v1, lane, E), axis=1, keepdims=True)
    rw2 = jnp.where(lane == i1, -jnp.inf, rw)
    v2 = jnp.max(rw2, axis=1, keepdims=True)
    i2 = jnp.min(jnp.where(rw2 == v2, lane, E), axis=1, keepdims=True)
    s = v1 + v2
    i1_ref[...] = i1
    i2_ref[...] = i2
    w1n_ref[...] = v1 / s
    w2n_ref[...] = v2 / s


def _sc_gather_body(x_hbm, tok_hbm, perm_hbm, fw_hbm, xs_hbm, ws_hbm,
                    tok_v, perm_v, w_v, rows_v, sem):
    wid = lax.axis_index("s") * NC + lax.axis_index("c")
    ch = A // NW
    sub = 64
    base = wid * ch
    pltpu.sync_copy(tok_hbm.at[pl.ds(base, ch)], tok_v)
    pltpu.sync_copy(perm_hbm.at[pl.ds(base, ch)], perm_v)
    for sc in range(ch // sub):
        pltpu.async_copy(x_hbm.at[tok_v.at[pl.ds(sc * sub, sub)]],
                         rows_v, sem).wait()
        pltpu.sync_copy(rows_v, xs_hbm.at[pl.ds(base + sc * sub, sub)])
        pltpu.async_copy(fw_hbm.at[perm_v.at[pl.ds(sc * sub, sub)]],
                         w_v.at[pl.ds(sc * sub, sub)], sem).wait()
    pltpu.sync_copy(w_v, ws_hbm.at[pl.ds(base, ch)])


def _sc_gather(x, sorted_token, perm, flat_w):
    ch = A // NW
    mesh = plsc.VectorSubcoreMesh(core_axis_name="c", subcore_axis_name="s")
    return pl.kernel(
        _sc_gather_body,
        out_type=(jax.ShapeDtypeStruct((A, H), jnp.float32),
                  jax.ShapeDtypeStruct((A,), jnp.float32)),
        mesh=mesh,
        scratch_types=[
            pltpu.VMEM((ch,), jnp.int32),
            pltpu.VMEM((ch,), jnp.int32),
            pltpu.VMEM((ch,), jnp.float32),
            pltpu.VMEM((64, H), jnp.float32),
            pltpu.SemaphoreType.DMA,
        ],
    )(x, sorted_token, perm, flat_w)


def _ffn_kernel(mt_ref, eid_ref, lo_ref, hi_ref, first_ref,
                xs_ref, w1_ref, w3_ref, w2_ref, ws_ref, out_ref):
    i = pl.program_id(0)

    @pl.when(first_ref[i] == 1)
    def _init():
        out_ref[...] = jnp.zeros_like(out_ref)

    mt = mt_ref[i]
    lo = lo_ref[i] - mt * TM
    hi = hi_ref[i] - mt * TM
    r = lax.broadcasted_iota(jnp.int32, (TM, 1), 0)
    wmask = jnp.where((r >= lo) & (r < hi), ws_ref[...], 0.0)

    x = xs_ref[...].astype(jnp.bfloat16)
    acc = jnp.zeros((TM, H), jnp.float32)
    for fb in range(F // FB):
        w1b = w1_ref[0, pl.ds(fb * FB, FB), :].astype(jnp.bfloat16)
        w3b = w3_ref[0, pl.ds(fb * FB, FB), :].astype(jnp.bfloat16)
        w2b = w2_ref[0, :, pl.ds(fb * FB, FB)].astype(jnp.bfloat16)
        h1 = lax.dot_general(x, w1b, (((1,), (1,)), ((), ())),
                             preferred_element_type=jnp.float32)
        h3 = lax.dot_general(x, w3b, (((1,), (1,)), ((), ())),
                             preferred_element_type=jnp.float32)
        hm = (h1 * (1.0 / (1.0 + jnp.exp(-h1))) * h3).astype(jnp.bfloat16)
        acc = acc + lax.dot_general(hm, w2b, (((1,), (1,)), ((), ())),
                                    preferred_element_type=jnp.float32)
    out_ref[...] += acc * wmask


def _grouped_ffn(xs, w1, w3, w2, ws_col, mt, eid, row_lo, row_hi, first):
    grid_spec = pltpu.PrefetchScalarGridSpec(
        num_scalar_prefetch=5,
        grid=(G,),
        in_specs=[
            pl.BlockSpec((TM, H), lambda i, mt, eid, lo, hi, fst: (mt[i], 0)),
            pl.BlockSpec((1, F, H), lambda i, mt, eid, lo, hi, fst: (eid[i], 0, 0)),
            pl.BlockSpec((1, F, H), lambda i, mt, eid, lo, hi, fst: (eid[i], 0, 0)),
            pl.BlockSpec((1, H, F), lambda i, mt, eid, lo, hi, fst: (eid[i], 0, 0)),
            pl.BlockSpec((TM, 1), lambda i, mt, eid, lo, hi, fst: (mt[i], 0)),
        ],
        out_specs=pl.BlockSpec((TM, H), lambda i, mt, eid, lo, hi, fst: (mt[i], 0)),
    )
    return pl.pallas_call(
        _ffn_kernel,
        grid_spec=grid_spec,
        out_shape=jax.ShapeDtypeStruct((A, H), jnp.float32),
    )(mt, eid, row_lo, row_hi, first, xs, w1, w3, w2, ws_col)


def _sc_combine_body(ys_hbm, ip_hbm, out_hbm, ip_v, rows_v, o_v, sem):
    wid = lax.axis_index("s") * NC + lax.axis_index("c")
    tok_per = T // NW
    subt = 32
    base_tok = wid * tok_per
    pltpu.sync_copy(ip_hbm.at[pl.ds(TOP_K * base_tok, TOP_K * tok_per)], ip_v)
    for s in range(tok_per // subt):
        pltpu.async_copy(ys_hbm.at[ip_v.at[pl.ds(s * TOP_K * subt, TOP_K * subt)]],
                         rows_v, sem).wait()

        def body(k, _):
            for j in range(H // 16):
                sl = pl.ds(j * 16, 16)
                o_v[k, sl] = rows_v[2 * k, sl] + rows_v[2 * k + 1, sl]
            return 0

        lax.fori_loop(0, subt, body, 0)
        pltpu.sync_copy(o_v, out_hbm.at[pl.ds(base_tok + s * subt, subt)])


def _sc_combine(ys, inv_perm):
    mesh = plsc.VectorSubcoreMesh(core_axis_name="c", subcore_axis_name="s")
    tok_per = T // NW
    subt = 32
    return pl.kernel(
        _sc_combine_body,
        out_type=jax.ShapeDtypeStruct((T, H), jnp.float32),
        mesh=mesh,
        scratch_types=[
            pltpu.VMEM((TOP_K * tok_per,), jnp.int32),
            pltpu.VMEM((TOP_K * subt, H), jnp.float32),
            pltpu.VMEM((subt, H), jnp.float32),
            pltpu.SemaphoreType.DMA,
        ],
    )(ys, inv_perm)


@jax.jit
def kernel(hidden_states, gate_w, w1, w3, w2):
    b, s, h = hidden_states.shape
    x = hidden_states.reshape(-1, h)

    logits, i1, i2, w1n, w2n = pl.pallas_call(
        _router_kernel,
        out_shape=(
            jax.ShapeDtypeStruct((T, E), jnp.float32),
            jax.ShapeDtypeStruct((T, 1), jnp.int32),
            jax.ShapeDtypeStruct((T, 1), jnp.int32),
            jax.ShapeDtypeStruct((T, 1), jnp.float32),
            jax.ShapeDtypeStruct((T, 1), jnp.float32),
        ),
    )(x, gate_w)

    flat_e = jnp.concatenate([i1, i2], axis=1).reshape(-1)
    flat_w = jnp.concatenate([w1n, w2n], axis=1).reshape(-1)
    perm = jnp.argsort(flat_e, stable=True).astype(jnp.int32)
    inv_perm = jnp.zeros((A,), jnp.int32).at[perm].set(
        jnp.arange(A, dtype=jnp.int32))
    sorted_token = (perm // TOP_K).astype(jnp.int32)

    sorted_e = jnp.sort(flat_e)
    ends = jnp.searchsorted(sorted_e, jnp.arange(E, dtype=jnp.int32),
                            side="right").astype(jnp.int32)
    tile_starts = jnp.arange(NUM_TILES, dtype=jnp.int32) * TM
    cuts = jnp.sort(jnp.concatenate([tile_starts, ends[:-1]]))
    seg_lo = cuts
    seg_hi = jnp.concatenate([cuts[1:], jnp.array([A], jnp.int32)])
    eid = jnp.clip(jnp.searchsorted(ends, seg_lo, side="right"),
                   0, E - 1).astype(jnp.int32)
    mt = jnp.clip(seg_lo // TM, 0, NUM_TILES - 1).astype(jnp.int32)
    first = jnp.concatenate(
        [jnp.ones((1,), jnp.int32),
         (mt[1:] != mt[:-1]).astype(jnp.int32)])

    xs, ws = _sc_gather(x, sorted_token, perm, flat_w)
    ys = _grouped_ffn(xs, w1, w3, w2, ws.reshape(A, 1),
                      mt, eid, seg_lo, seg_hi, first)
    final = _sc_combine(ys, inv_perm)

    return (final, logits)


# speedup vs baseline: 5.4960x; 1.0085x over previous
import functools

import jax
import jax.numpy as jnp
from jax import lax
from jax.experimental import pallas as pl
from jax.experimental.pallas import tpu as pltpu
from jax.experimental.pallas import tpu_sc as plsc

E = 64
H = 768
F = 2048
TOP_K = 2

T = 4096
A = T * TOP_K
TM = 128
NUM_TILES = A // TM
G = NUM_TILES + E - 1
FB = 512

NC, NS = 2, 16
NW = NC * NS


def _router_kernel(x_ref, gw_ref, logits_ref, i1_ref, i2_ref, w1n_ref, w2n_ref):
    x = x_ref[...]
    gw = gw_ref[...]
    logits = lax.dot_general(
        x, gw, (((1,), (1,)), ((), ())), preferred_element_type=jnp.float32)
    logits_ref[...] = logits
    m = jnp.max(logits, axis=1, keepdims=True)
    ex = jnp.exp(logits - m)
    rw = ex / jnp.sum(ex, axis=1, keepdims=True)
    lane = lax.broadcasted_iota(jnp.int32, rw.shape, 1)
    v1 = jnp.max(rw, axis=1, keepdims=True)
    i1 = jnp.min(jnp.where(rw == v1, lane, E), axis=1, keepdims=True)
    rw2 = jnp.where(lane == i1, -jnp.inf, rw)
    v2 = jnp.max(rw2, axis=1, keepdims=True)
    i2 = jnp.min(jnp.where(rw2 == v2, lane, E), axis=1, keepdims=True)
    s = v1 + v2
    i1_ref[...] = i1
    i2_ref[...] = i2
    w1n_ref[...] = v1 / s
    w2n_ref[...] = v2 / s


def _sc_gather_body(x_hbm, tok_hbm, perm_hbm, fw_hbm, xs_hbm, ws_hbm,
                    tok_v, perm_v, w_v, rows_v, sem0, sem1, wsem):
    wid = lax.axis_index("s") * NC + lax.axis_index("c")
    ch = A // NW
    sub = 64
    n = ch // sub
    base = wid * ch
    sems = (sem0, sem1)
    pltpu.sync_copy(tok_hbm.at[pl.ds(base, ch)], tok_v)
    pltpu.sync_copy(perm_hbm.at[pl.ds(base, ch)], perm_v)
    whs = [pltpu.async_copy(fw_hbm.at[perm_v.at[pl.ds(j * 128, 128)]],
                            w_v.at[pl.ds(j * 128, 128)], wsem)
           for j in range(ch // 128)]
    prev = pltpu.async_copy(x_hbm.at[tok_v.at[pl.ds(0, sub)]],
                            rows_v.at[0], sems[0])
    for s in range(n):
        nxt = None
        if s + 1 < n:
            nxt = pltpu.async_copy(
                x_hbm.at[tok_v.at[pl.ds((s + 1) * sub, sub)]],
                rows_v.at[(s + 1) % 2], sems[(s + 1) % 2])
        prev.wait()
        pltpu.sync_copy(rows_v.at[s % 2], xs_hbm.at[pl.ds(base + s * sub, sub)])
        prev = nxt
    for wh in whs:
        wh.wait()
    pltpu.sync_copy(w_v, ws_hbm.at[pl.ds(base, ch)])


def _sc_gather(x, sorted_token, perm, flat_w):
    ch = A // NW
    mesh = plsc.VectorSubcoreMesh(core_axis_name="c", subcore_axis_name="s")
    return pl.kernel(
        _sc_gather_body,
        out_type=(jax.ShapeDtypeStruct((A, H), jnp.float32),
                  jax.ShapeDtypeStruct((A,), jnp.float32)),
        mesh=mesh,
        scratch_types=[
            pltpu.VMEM((ch,), jnp.int32),
            pltpu.VMEM((ch,), jnp.int32),
            pltpu.VMEM((ch,), jnp.float32),
            pltpu.VMEM((2, 64, H), jnp.float32),
            pltpu.SemaphoreType.DMA,
            pltpu.SemaphoreType.DMA,
            pltpu.SemaphoreType.DMA,
        ],
    )(x, sorted_token, perm, flat_w)


def _ffn_kernel(mt_ref, eid_ref, lo_ref, hi_ref, first_ref,
                xs_ref, w1_ref, w3_ref, w2_ref, ws_ref, out_ref):
    i = pl.program_id(0)

    @pl.when(first_ref[i] == 1)
    def _init():
        out_ref[...] = jnp.zeros_like(out_ref)

    mt = mt_ref[i]
    lo = lo_ref[i] - mt * TM
    hi = hi_ref[i] - mt * TM
    r = lax.broadcasted_iota(jnp.int32, (TM, 1), 0)
    wmask = jnp.where((r >= lo) & (r < hi), ws_ref[...], 0.0)

    x = xs_ref[...].astype(jnp.bfloat16)
    acc = jnp.zeros((TM, H), jnp.float32)
    for fb in range(F // FB):
        w1b = w1_ref[0, pl.ds(fb * FB, FB), :].astype(jnp.bfloat16)
        w3b = w3_ref[0, pl.ds(fb * FB, FB), :].astype(jnp.bfloat16)
        w2b = w2_ref[0, :, pl.ds(fb * FB, FB)].astype(jnp.bfloat16)
        h1 = lax.dot_general(x, w1b, (((1,), (1,)), ((), ())),
                             preferred_element_type=jnp.float32)
        h3 = lax.dot_general(x, w3b, (((1,), (1,)), ((), ())),
                             preferred_element_type=jnp.float32)
        hm = (h1 * (1.0 / (1.0 + jnp.exp(-h1))) * h3).astype(jnp.bfloat16)
        acc = acc + lax.dot_general(hm, w2b, (((1,), (1,)), ((), ())),
                                    preferred_element_type=jnp.float32)
    out_ref[...] += acc * wmask


def _grouped_ffn(xs, w1, w3, w2, ws_col, mt, eid, row_lo, row_hi, first):
    grid_spec = pltpu.PrefetchScalarGridSpec(
        num_scalar_prefetch=5,
        grid=(G,),
        in_specs=[
            pl.BlockSpec((TM, H), lambda i, mt, eid, lo, hi, fst: (mt[i], 0)),
            pl.BlockSpec((1, F, H), lambda i, mt, eid, lo, hi, fst: (eid[i], 0, 0)),
            pl.BlockSpec((1, F, H), lambda i, mt, eid, lo, hi, fst: (eid[i], 0, 0)),
            pl.BlockSpec((1, H, F), lambda i, mt, eid, lo, hi, fst: (eid[i], 0, 0)),
            pl.BlockSpec((TM, 1), lambda i, mt, eid, lo, hi, fst: (mt[i], 0)),
        ],
        out_specs=pl.BlockSpec((TM, H), lambda i, mt, eid, lo, hi, fst: (mt[i], 0)),
    )
    return pl.pallas_call(
        _ffn_kernel,
        grid_spec=grid_spec,
        out_shape=jax.ShapeDtypeStruct((A, H), jnp.float32),
    )(mt, eid, row_lo, row_hi, first, xs, w1, w3, w2, ws_col)


def _sc_combine_body(ys_hbm, ip_hbm, out_hbm, ip_v, rows_v, o_v, sem0, sem1):
    wid = lax.axis_index("s") * NC + lax.axis_index("c")
    tok_per = T // NW
    subt = 32
    n = tok_per // subt
    base_tok = wid * tok_per
    sems = (sem0, sem1)
    pltpu.sync_copy(ip_hbm.at[pl.ds(TOP_K * base_tok, TOP_K * tok_per)], ip_v)
    prev = pltpu.async_copy(
        ys_hbm.at[ip_v.at[pl.ds(0, TOP_K * subt)]], rows_v.at[0], sems[0])
    for s in range(n):
        nxt = None
        if s + 1 < n:
            nxt = pltpu.async_copy(
                ys_hbm.at[ip_v.at[pl.ds((s + 1) * TOP_K * subt, TOP_K * subt)]],
                rows_v.at[(s + 1) % 2], sems[(s + 1) % 2])
        prev.wait()
        rbuf = rows_v.at[s % 2]

        def body(k, _):
            for j in range(H // 16):
                sl = pl.ds(j * 16, 16)
                o_v[k, sl] = rbuf[2 * k, sl] + rbuf[2 * k + 1, sl]
            return 0

        lax.fori_loop(0, subt, body, 0)
        pltpu.sync_copy(o_v, out_hbm.at[pl.ds(base_tok + s * subt, subt)])
        prev = nxt


def _sc_combine(ys, inv_perm):
    mesh = plsc.VectorSubcoreMesh(core_axis_name="c", subcore_axis_name="s")
    tok_per = T // NW
    subt = 32
    return pl.kernel(
        _sc_combine_body,
        out_type=jax.ShapeDtypeStruct((T, H), jnp.float32),
        mesh=mesh,
        scratch_types=[
            pltpu.VMEM((TOP_K * tok_per,), jnp.int32),
            pltpu.VMEM((2, TOP_K * subt, H), jnp.float32),
            pltpu.VMEM((subt, H), jnp.float32),
            pltpu.SemaphoreType.DMA,
            pltpu.SemaphoreType.DMA,
        ],
    )(ys, inv_perm)


@jax.jit
def kernel(hidden_states, gate_w, w1, w3, w2):
    b, s, h = hidden_states.shape
    x = hidden_states.reshape(-1, h)

    logits, i1, i2, w1n, w2n = pl.pallas_call(
        _router_kernel,
        out_shape=(
            jax.ShapeDtypeStruct((T, E), jnp.float32),
            jax.ShapeDtypeStruct((T, 1), jnp.int32),
            jax.ShapeDtypeStruct((T, 1), jnp.int32),
            jax.ShapeDtypeStruct((T, 1), jnp.float32),
            jax.ShapeDtypeStruct((T, 1), jnp.float32),
        ),
    )(x, gate_w)

    flat_e = jnp.concatenate([i1, i2], axis=1).reshape(-1)
    flat_w = jnp.concatenate([w1n, w2n], axis=1).reshape(-1)
    perm = jnp.argsort(flat_e, stable=True).astype(jnp.int32)
    inv_perm = jnp.zeros((A,), jnp.int32).at[perm].set(
        jnp.arange(A, dtype=jnp.int32))
    sorted_token = (perm // TOP_K).astype(jnp.int32)

    sorted_e = jnp.sort(flat_e)
    ends = jnp.searchsorted(sorted_e, jnp.arange(E, dtype=jnp.int32),
                            side="right").astype(jnp.int32)
    tile_starts = jnp.arange(NUM_TILES, dtype=jnp.int32) * TM
    cuts = jnp.sort(jnp.concatenate([tile_starts, ends[:-1]]))
    seg_lo = cuts
    seg_hi = jnp.concatenate([cuts[1:], jnp.array([A], jnp.int32)])
    eid = jnp.clip(jnp.searchsorted(ends, seg_lo, side="right"),
                   0, E - 1).astype(jnp.int32)
    mt = jnp.clip(seg_lo // TM, 0, NUM_TILES - 1).astype(jnp.int32)
    first = jnp.concatenate(
        [jnp.ones((1,), jnp.int32),
         (mt[1:] != mt[:-1]).astype(jnp.int32)])

    xs, ws = _sc_gather(x, sorted_token, perm, flat_w)
    ys = _grouped_ffn(xs, w1, w3, w2, ws.reshape(A, 1),
                      mt, eid, seg_lo, seg_hi, first)
    final = _sc_combine(ys, inv_perm)

    return (final, logits)


# speedup vs baseline: 6.8078x; 1.2387x over previous
import functools

import jax
import jax.numpy as jnp
from jax import lax
from jax.experimental import pallas as pl
from jax.experimental.pallas import tpu as pltpu
from jax.experimental.pallas import tpu_sc as plsc

E = 64
H = 768
F = 2048
TOP_K = 2

T = 4096
A = T * TOP_K
TM = 512
NUM_TILES = A // TM
G = NUM_TILES + E - 1
FB = 512

NC, NS = 2, 16
NW = NC * NS


def _router_kernel(x_ref, gw_ref, logits_ref, i1_ref, i2_ref, w1n_ref, w2n_ref):
    x = x_ref[...]
    gw = gw_ref[...]
    logits = lax.dot_general(
        x, gw, (((1,), (1,)), ((), ())), preferred_element_type=jnp.float32)
    logits_ref[...] = logits
    m = jnp.max(logits, axis=1, keepdims=True)
    ex = jnp.exp(logits - m)
    rw = ex / jnp.sum(ex, axis=1, keepdims=True)
    lane = lax.broadcasted_iota(jnp.int32, rw.shape, 1)
    v1 = jnp.max(rw, axis=1, keepdims=True)
    i1 = jnp.min(jnp.where(rw == v1, lane, E), axis=1, keepdims=True)
    rw2 = jnp.where(lane == i1, -jnp.inf, rw)
    v2 = jnp.max(rw2, axis=1, keepdims=True)
    i2 = jnp.min(jnp.where(rw2 == v2, lane, E), axis=1, keepdims=True)
    s = v1 + v2
    i1_ref[...] = i1
    i2_ref[...] = i2
    w1n_ref[...] = v1 / s
    w2n_ref[...] = v2 / s


def _sc_gather_body(x_hbm, tok_hbm, perm_hbm, fw_hbm, xs_hbm, ws_hbm,
                    tok_v, perm_v, w_v, rows_v, sem0, sem1, wsem):
    wid = lax.axis_index("s") * NC + lax.axis_index("c")
    ch = A // NW
    sub = 64
    n = ch // sub
    base = wid * ch
    sems = (sem0, sem1)
    pltpu.sync_copy(tok_hbm.at[pl.ds(base, ch)], tok_v)
    pltpu.sync_copy(perm_hbm.at[pl.ds(base, ch)], perm_v)
    whs = [pltpu.async_copy(fw_hbm.at[perm_v.at[pl.ds(j * 128, 128)]],
                            w_v.at[pl.ds(j * 128, 128)], wsem)
           for j in range(ch // 128)]
    prev = pltpu.async_copy(x_hbm.at[tok_v.at[pl.ds(0, sub)]],
                            rows_v.at[0], sems[0])
    for s in range(n):
        nxt = None
        if s + 1 < n:
            nxt = pltpu.async_copy(
                x_hbm.at[tok_v.at[pl.ds((s + 1) * sub, sub)]],
                rows_v.at[(s + 1) % 2], sems[(s + 1) % 2])
        prev.wait()
        pltpu.sync_copy(rows_v.at[s % 2], xs_hbm.at[pl.ds(base + s * sub, sub)])
        prev = nxt
    for wh in whs:
        wh.wait()
    pltpu.sync_copy(w_v, ws_hbm.at[pl.ds(base, ch)])


def _sc_gather(x, sorted_token, perm, flat_w):
    ch = A // NW
    mesh = plsc.VectorSubcoreMesh(core_axis_name="c", subcore_axis_name="s")
    return pl.kernel(
        _sc_gather_body,
        out_type=(jax.ShapeDtypeStruct((A, H), jnp.float32),
                  jax.ShapeDtypeStruct((A,), jnp.float32)),
        mesh=mesh,
        scratch_types=[
            pltpu.VMEM((ch,), jnp.int32),
            pltpu.VMEM((ch,), jnp.int32),
            pltpu.VMEM((ch,), jnp.float32),
            pltpu.VMEM((2, 64, H), jnp.float32),
            pltpu.SemaphoreType.DMA,
            pltpu.SemaphoreType.DMA,
            pltpu.SemaphoreType.DMA,
        ],
    )(x, sorted_token, perm, flat_w)


def _ffn_kernel(mt_ref, eid_ref, lo_ref, hi_ref, first_ref,
                xs_ref, w1_ref, w3_ref, w2_ref, ws_ref, out_ref):
    i = pl.program_id(0)

    @pl.when(first_ref[i] == 1)
    def _init():
        out_ref[...] = jnp.zeros_like(out_ref)

    mt = mt_ref[i]
    lo = lo_ref[i] - mt * TM
    hi = hi_ref[i] - mt * TM
    r = lax.broadcasted_iota(jnp.int32, (TM, 1), 0)
    wmask = jnp.where((r >= lo) & (r < hi), ws_ref[...], 0.0)

    x = xs_ref[...]
    acc = jnp.zeros((TM, H), jnp.float32)
    for fb in range(F // FB):
        w1b = w1_ref[0, pl.ds(fb * FB, FB), :]
        w3b = w3_ref[0, pl.ds(fb * FB, FB), :]
        w2b = w2_ref[0, :, pl.ds(fb * FB, FB)]
        h1 = lax.dot_general(x, w1b, (((1,), (1,)), ((), ())),
                             preferred_element_type=jnp.float32)
        h3 = lax.dot_general(x, w3b, (((1,), (1,)), ((), ())),
                             preferred_element_type=jnp.float32)
        hm = h1 * (1.0 / (1.0 + jnp.exp(-h1))) * h3
        acc = acc + lax.dot_general(hm, w2b, (((1,), (1,)), ((), ())),
                                    preferred_element_type=jnp.float32)
    out_ref[...] += acc * wmask


def _grouped_ffn(xs, w1, w3, w2, ws_col, mt, eid, row_lo, row_hi, first):
    grid_spec = pltpu.PrefetchScalarGridSpec(
        num_scalar_prefetch=5,
        grid=(G,),
        in_specs=[
            pl.BlockSpec((TM, H), lambda i, mt, eid, lo, hi, fst: (mt[i], 0)),
            pl.BlockSpec((1, F, H), lambda i, mt, eid, lo, hi, fst: (eid[i], 0, 0)),
            pl.BlockSpec((1, F, H), lambda i, mt, eid, lo, hi, fst: (eid[i], 0, 0)),
            pl.BlockSpec((1, H, F), lambda i, mt, eid, lo, hi, fst: (eid[i], 0, 0)),
            pl.BlockSpec((TM, 1), lambda i, mt, eid, lo, hi, fst: (mt[i], 0)),
        ],
        out_specs=pl.BlockSpec((TM, H), lambda i, mt, eid, lo, hi, fst: (mt[i], 0)),
    )
    return pl.pallas_call(
        _ffn_kernel,
        grid_spec=grid_spec,
        out_shape=jax.ShapeDtypeStruct((A, H), jnp.float32),
    )(mt, eid, row_lo, row_hi, first, xs, w1, w3, w2, ws_col)


def _sc_combine_body(ys_hbm, ip_hbm, out_hbm, ip_v, rows_v, o_v, sem0, sem1):
    wid = lax.axis_index("s") * NC + lax.axis_index("c")
    tok_per = T // NW
    subt = 32
    n = tok_per // subt
    base_tok = wid * tok_per
    sems = (sem0, sem1)
    pltpu.sync_copy(ip_hbm.at[pl.ds(TOP_K * base_tok, TOP_K * tok_per)], ip_v)
    prev = pltpu.async_copy(
        ys_hbm.at[ip_v.at[pl.ds(0, TOP_K * subt)]], rows_v.at[0], sems[0])
    for s in range(n):
        nxt = None
        if s + 1 < n:
            nxt = pltpu.async_copy(
                ys_hbm.at[ip_v.at[pl.ds((s + 1) * TOP_K * subt, TOP_K * subt)]],
                rows_v.at[(s + 1) % 2], sems[(s + 1) % 2])
        prev.wait()
        rbuf = rows_v.at[s % 2]

        def body(k, _):
            for j in range(H // 16):
                sl = pl.ds(j * 16, 16)
                o_v[k, sl] = rbuf[2 * k, sl] + rbuf[2 * k + 1, sl]
            return 0

        lax.fori_loop(0, subt, body, 0)
        pltpu.sync_copy(o_v, out_hbm.at[pl.ds(base_tok + s * subt, subt)])
        prev = nxt


def _sc_combine(ys, inv_perm):
    mesh = plsc.VectorSubcoreMesh(core_axis_name="c", subcore_axis_name="s")
    tok_per = T // NW
    subt = 32
    return pl.kernel(
        _sc_combine_body,
        out_type=jax.ShapeDtypeStruct((T, H), jnp.float32),
        mesh=mesh,
        scratch_types=[
            pltpu.VMEM((TOP_K * tok_per,), jnp.int32),
            pltpu.VMEM((2, TOP_K * subt, H), jnp.float32),
            pltpu.VMEM((subt, H), jnp.float32),
            pltpu.SemaphoreType.DMA,
            pltpu.SemaphoreType.DMA,
        ],
    )(ys, inv_perm)


@jax.jit
def kernel(hidden_states, gate_w, w1, w3, w2):
    b, s, h = hidden_states.shape
    x = hidden_states.reshape(-1, h)

    logits, i1, i2, w1n, w2n = pl.pallas_call(
        _router_kernel,
        out_shape=(
            jax.ShapeDtypeStruct((T, E), jnp.float32),
            jax.ShapeDtypeStruct((T, 1), jnp.int32),
            jax.ShapeDtypeStruct((T, 1), jnp.int32),
            jax.ShapeDtypeStruct((T, 1), jnp.float32),
            jax.ShapeDtypeStruct((T, 1), jnp.float32),
        ),
    )(x, gate_w)

    flat_e = jnp.concatenate([i1, i2], axis=1).reshape(-1)
    flat_w = jnp.concatenate([w1n, w2n], axis=1).reshape(-1)
    perm = jnp.argsort(flat_e, stable=True).astype(jnp.int32)
    inv_perm = jnp.zeros((A,), jnp.int32).at[perm].set(
        jnp.arange(A, dtype=jnp.int32))
    sorted_token = (perm // TOP_K).astype(jnp.int32)

    sorted_e = jnp.sort(flat_e)
    ends = jnp.searchsorted(sorted_e, jnp.arange(E, dtype=jnp.int32),
                            side="right").astype(jnp.int32)
    tile_starts = jnp.arange(NUM_TILES, dtype=jnp.int32) * TM
    cuts = jnp.sort(jnp.concatenate([tile_starts, ends[:-1]]))
    seg_lo = cuts
    seg_hi = jnp.concatenate([cuts[1:], jnp.array([A], jnp.int32)])
    eid = jnp.clip(jnp.searchsorted(ends, seg_lo, side="right"),
                   0, E - 1).astype(jnp.int32)
    mt = jnp.clip(seg_lo // TM, 0, NUM_TILES - 1).astype(jnp.int32)
    first = jnp.concatenate(
        [jnp.ones((1,), jnp.int32),
         (mt[1:] != mt[:-1]).astype(jnp.int32)])

    xs, ws = _sc_gather(x, sorted_token, perm, flat_w)
    ys = _grouped_ffn(xs, w1, w3, w2, ws.reshape(A, 1),
                      mt, eid, seg_lo, seg_hi, first)
    final = _sc_combine(ys, inv_perm)

    return (final, logits)
